# baseline (speedup 1.0000x reference)
import functools

import jax
import jax.numpy as jnp
from jax import lax
from jax.experimental import pallas as pl
from jax.experimental.pallas import tpu as pltpu
from jax.experimental.pallas import tpu_sc as plsc

VOCAB = 100000
WORLD = 8
LOCAL_VOCAB = VOCAB // WORLD
HIDDEN = 1024
MAXSEQ = 2048
BATCH = 4
NTOK = BATCH * MAXSEQ

NC, NS, LANES = 2, 16, 16
NW = NC * NS
TPW = NTOK // NW
CHUNK = 16
NSLOT = 3
NCHUNK = TPW // CHUNK
NSTEP = (NCHUNK - 1) // NSLOT
NVREG = HIDDEN // LANES

_mesh = plsc.VectorSubcoreMesh(core_axis_name="c", subcore_axis_name="s")


@functools.partial(
    pl.kernel,
    out_type=jax.ShapeDtypeStruct((NTOK, HIDDEN), jnp.float32),
    mesh=_mesh,
    scratch_types=[
        pltpu.VMEM((TPW,), jnp.int32),
        pltpu.VMEM((1, HIDDEN), jnp.float32),
        pltpu.VMEM((CHUNK, HIDDEN), jnp.float32),
        pltpu.VMEM((CHUNK, HIDDEN), jnp.float32),
        pltpu.VMEM((CHUNK, HIDDEN), jnp.float32),
        pltpu.VMEM((CHUNK, HIDDEN), jnp.float32),
        pltpu.VMEM((CHUNK, HIDDEN), jnp.float32),
        pltpu.VMEM((CHUNK, HIDDEN), jnp.float32),
        pltpu.SMEM((CHUNK + 1,), jnp.int32),
        pltpu.SMEM((CHUNK + 1,), jnp.int32),
        pltpu.SMEM((CHUNK + 1,), jnp.int32),
        pltpu.SemaphoreType.DMA,
        pltpu.SemaphoreType.DMA,
        pltpu.SemaphoreType.DMA,
        pltpu.SemaphoreType.DMA,
        pltpu.SemaphoreType.DMA,
        pltpu.SemaphoreType.DMA,
        pltpu.SemaphoreType.DMA,
        pltpu.SemaphoreType.DMA,
        pltpu.SemaphoreType.DMA,
    ],
)
def _embed(ids_hbm, word_hbm, pos_hbm, out_hbm, idx_v, row0, wbuf0, wbuf1,
           wbuf2, gbuf0, gbuf1, gbuf2, msm0, msm1, msm2, gsem0, gsem1,
           gsem2, psem0, psem1, psem2, ssem0, ssem1, ssem2):
    wid = lax.axis_index("s") * NC + lax.axis_index("c")
    base = wid * TPW
    pos_base = base % MAXSEQ

    pltpu.sync_copy(ids_hbm.at[pl.ds(base, TPW)], idx_v)
    pltpu.sync_copy(word_hbm.at[pl.ds(0, 1)], row0)

    wbufs = (wbuf0, wbuf1, wbuf2)
    gbufs = (gbuf0, gbuf1, gbuf2)
    msms = (msm0, msm1, msm2)
    gsems = (gsem0, gsem1, gsem2)
    psems = (psem0, psem1, psem2)
    ssems = (ssem0, ssem1, ssem2)

    def drain(src, dst, sem):
        pltpu.make_async_copy(src, dst, sem).wait()

    def issue_chunk(ci, b):
        vv = idx_v[pl.ds(ci * CHUNK, LANES)]
        n_in = jnp.int32(0)
        for r in range(CHUNK):
            rid = lax.squeeze(lax.slice(vv, (r,), (r + 1,)), (0,))
            in_shard = rid < LOCAL_VOCAB
            msms[b][r] = jnp.where(in_shard, 1, 0).astype(jnp.int32)
            n_in = n_in + jnp.where(in_shard, 1, 0).astype(jnp.int32)

            @pl.when(in_shard)
            def _():
                pltpu.async_copy(word_hbm.at[pl.ds(rid, 1)],
                                 gbufs[b].at[pl.ds(r, 1)], gsems[b])

        msms[b][CHUNK] = n_in
        pltpu.async_copy(pos_hbm.at[pl.ds(pos_base + ci * CHUNK, CHUNK)],
                         wbufs[b], psems[b])

    def finish_chunk(ci, b):
        drain(pos_hbm.at[pl.ds(0, CHUNK)], wbufs[b], psems[b])

        def col_body(u, _):
            sl = pl.ds(u * LANES, LANES)
            vr0 = row0[0, sl]
            for r in range(CHUNK):
                plsc.addupdate(wbufs[b].at[r, sl], vr0)
            return 0

        lax.fori_loop(0, NVREG, col_body, 0)

        def drain_body(_, acc):
            drain(word_hbm.at[pl.ds(0, 1)], gbufs[b].at[pl.ds(0, 1)],
                  gsems[b])
            return acc

        lax.fori_loop(0, msms[b][CHUNK], drain_body, 0)

        def row_body(r, _):
            @pl.when(msms[b][r] != 0)
            def _():
                for u in range(NVREG):
                    sl = pl.ds(u * LANES, LANES)
                    vr0 = row0[0, sl]
                    wbufs[b][r, sl] = gbufs[b][r, sl] + (wbufs[b][r, sl] -
                                                         vr0)

            return 0

        lax.fori_loop(0, CHUNK, row_body, 0)
        pltpu.async_copy(wbufs[b], out_hbm.at[pl.ds(base + ci * CHUNK, CHUNK)],
                         ssems[b])

    def step(k, _):
        for b in range(NSLOT):
            ci = k * NSLOT + b

            @pl.when(k > 0)
            def _():
                drain(wbufs[b], out_hbm.at[pl.ds(0, CHUNK)], ssems[b])

            issue_chunk(ci, b)
        for b in range(NSLOT):
            finish_chunk(k * NSLOT + b, b)
        return 0

    lax.fori_loop(0, NSTEP, step, 0)
    drain(wbufs[0], out_hbm.at[pl.ds(0, CHUNK)], ssems[0])
    issue_chunk(NCHUNK - 1, 0)
    finish_chunk(NCHUNK - 1, 0)
    for b in range(NSLOT):
        drain(wbufs[b], out_hbm.at[pl.ds(0, CHUNK)], ssems[b])


def kernel(input_ids, word_table, pos_table):
    ids_flat = input_ids.reshape(NTOK)
    out = _embed(ids_flat, word_table, pos_table)
    return out.reshape(BATCH, MAXSEQ, HIDDEN)


# speedup vs baseline: 1.0460x; 1.0460x over previous
import functools

import jax
import jax.numpy as jnp
from jax import lax
from jax.experimental import pallas as pl
from jax.experimental.pallas import tpu as pltpu
from jax.experimental.pallas import tpu_sc as plsc

VOCAB = 100000
WORLD = 8
LOCAL_VOCAB = VOCAB // WORLD
HIDDEN = 1024
MAXSEQ = 2048
BATCH = 4
NTOK = BATCH * MAXSEQ

NC, NS, LANES = 2, 16, 16
NW = NC * NS
TPW = NTOK // NW
CHUNK = 16
NSLOT = 2
NCHUNK = TPW // CHUNK
NSTEP = NCHUNK // NSLOT
NVREG = HIDDEN // LANES

_mesh = plsc.VectorSubcoreMesh(core_axis_name="c", subcore_axis_name="s")


@functools.partial(
    pl.kernel,
    out_type=jax.ShapeDtypeStruct((NTOK, HIDDEN), jnp.float32),
    mesh=_mesh,
    scratch_types=[
        pltpu.VMEM((TPW,), jnp.int32),
        pltpu.VMEM((1, HIDDEN), jnp.float32),
        pltpu.VMEM((CHUNK, HIDDEN), jnp.float32),
        pltpu.VMEM((CHUNK, HIDDEN), jnp.float32),
        pltpu.VMEM((CHUNK, HIDDEN), jnp.float32),
        pltpu.VMEM((CHUNK, HIDDEN), jnp.float32),
        pltpu.SMEM((CHUNK + 1,), jnp.int32),
        pltpu.SMEM((CHUNK + 1,), jnp.int32),
        pltpu.SemaphoreType.DMA,
        pltpu.SemaphoreType.DMA,
        pltpu.SemaphoreType.DMA,
        pltpu.SemaphoreType.DMA,
        pltpu.SemaphoreType.DMA,
        pltpu.SemaphoreType.DMA,
    ],
)
def _embed(ids_hbm, word_hbm, pos_hbm, out_hbm, idx_v, row0, wbuf0, wbuf1,
           gbuf0, gbuf1, msm0, msm1, gsem0, gsem1, psem0, psem1, ssem0,
           ssem1):
    wid = lax.axis_index("s") * NC + lax.axis_index("c")
    base = wid * TPW
    pos_base = base % MAXSEQ

    pltpu.sync_copy(ids_hbm.at[pl.ds(base, TPW)], idx_v)
    pltpu.sync_copy(word_hbm.at[pl.ds(0, 1)], row0)

    wbufs = (wbuf0, wbuf1)
    gbufs = (gbuf0, gbuf1)
    msms = (msm0, msm1)
    gsems = (gsem0, gsem1)
    psems = (psem0, psem1)
    ssems = (ssem0, ssem1)

    def drain(src, dst, sem):
        pltpu.make_async_copy(src, dst, sem).wait()

    def issue_chunk(ci, b):
        vv = idx_v[pl.ds(ci * CHUNK, LANES)]
        n_in = jnp.int32(0)
        for r in range(CHUNK):
            rid = lax.squeeze(lax.slice(vv, (r,), (r + 1,)), (0,))
            in_shard = rid < LOCAL_VOCAB
            msms[b][r] = jnp.where(in_shard, 1, 0).astype(jnp.int32)
            n_in = n_in + jnp.where(in_shard, 1, 0).astype(jnp.int32)

            @pl.when(in_shard)
            def _():
                pltpu.async_copy(word_hbm.at[pl.ds(rid, 1)],
                                 gbufs[b].at[pl.ds(r, 1)], gsems[b])

        msms[b][CHUNK] = n_in
        pltpu.async_copy(pos_hbm.at[pl.ds(pos_base + ci * CHUNK, CHUNK)],
                         wbufs[b], psems[b])

    def finish_chunk(ci, b):
        drain(pos_hbm.at[pl.ds(0, CHUNK)], wbufs[b], psems[b])

        def col_body(u, _):
            sl = pl.ds(u * LANES, LANES)
            vr0 = row0[0, sl]
            for r in range(CHUNK):
                plsc.addupdate(wbufs[b].at[r, sl], vr0)
            return 0

        lax.fori_loop(0, NVREG, col_body, 0)

        def drain_body(_, acc):
            drain(word_hbm.at[pl.ds(0, 1)], gbufs[b].at[pl.ds(0, 1)],
                  gsems[b])
            return acc

        lax.fori_loop(0, msms[b][CHUNK], drain_body, 0)

        def row_body(r, _):
            @pl.when(msms[b][r] != 0)
            def _():
                for u in range(NVREG):
                    sl = pl.ds(u * LANES, LANES)
                    vr0 = row0[0, sl]
                    wbufs[b][r, sl] = gbufs[b][r, sl] + (wbufs[b][r, sl] -
                                                         vr0)

            return 0

        lax.fori_loop(0, CHUNK, row_body, 0)
        pltpu.async_copy(wbufs[b], out_hbm.at[pl.ds(base + ci * CHUNK, CHUNK)],
                         ssems[b])

    def step(k, _):
        for b in range(NSLOT):
            ci = k * NSLOT + b

            @pl.when(k > 0)
            def _():
                drain(wbufs[b], out_hbm.at[pl.ds(0, CHUNK)], ssems[b])

            issue_chunk(ci, b)
        for b in range(NSLOT):
            finish_chunk(k * NSLOT + b, b)
        return 0

    lax.fori_loop(0, NSTEP, step, 0)
    for b in range(NSLOT):
        drain(wbufs[b], out_hbm.at[pl.ds(0, CHUNK)], ssems[b])


def kernel(input_ids, word_table, pos_table):
    ids_flat = input_ids.reshape(NTOK)
    out = _embed(ids_flat, word_table, pos_table)
    return out.reshape(BATCH, MAXSEQ, HIDDEN)
